# Initial kernel scaffold; baseline (speedup 1.0000x reference)
import jax
import jax.numpy as jnp
from jax.experimental import pallas as pl


def kernel(x, idx, vals):
    raise NotImplementedError("write your pallas kernel here")


# speedup vs baseline: 1.7245x; 1.7245x over previous
import functools

import jax
import jax.numpy as jnp
from jax import lax
from jax.experimental import pallas as pl
from jax.experimental.pallas import tpu as pltpu
from jax.experimental.pallas import tpu_sc as plsc

L = 16
NC, NS = 2, 16
NW = NC * NS
R = 1_000_000
D = 64
B = 16384
RPW = R // NW
NCHUNK = B // L

_mesh = plsc.VectorSubcoreMesh(core_axis_name="c", subcore_axis_name="s")


@functools.partial(
    pl.kernel,
    out_type=(),
    mesh=_mesh,
    compiler_params=pltpu.CompilerParams(
        needs_layout_passes=False, use_tc_tiling_on_sc=False),
    scratch_types=[
        pltpu.VMEM((B,), jnp.int32),
        pltpu.VMEM((RPW,), jnp.int32),
        pltpu.VMEM((B + L,), jnp.int32),
        pltpu.VMEM((B + L,), jnp.int32),
        pltpu.VMEM((L, D), jnp.float32),
        pltpu.SemaphoreType.DMA,
    ],
)
def _sc_index_put(out_ref, idx_hbm, vals_hbm, idx_v, marker, plist, tlist,
                  rows, sem):
    wid = lax.axis_index("s") * NC + lax.axis_index("c")
    base = wid * RPW
    lane = lax.iota(jnp.int32, L)

    pltpu.sync_copy(idx_hbm, idx_v)

    def chunk(c):
        i16 = idx_v[pl.ds(c * L, L)]
        member = (i16 >= base) & (i16 < base + RPW)
        return i16, member, i16 - base, c * L + lane

    def p1(c, carry):
        _, member, local, pos = chunk(c)
        plsc.store_scatter(marker, [local], pos, mask=member)
        return carry

    lax.fori_loop(0, NCHUNK, p1, 0)

    def p2(c, cnt):
        i16, member, local, pos = chunk(c)
        g = plsc.load_gather(marker, [local], mask=member)
        win = member & (g == pos)
        plsc.store_compressed(plist.at[pl.ds(cnt, L)], pos, mask=win)
        plsc.store_compressed(tlist.at[pl.ds(cnt, L)], i16, mask=win)
        return cnt + jnp.sum(win.astype(jnp.int32))

    cnt = lax.fori_loop(0, NCHUNK, p2, jnp.int32(0))

    @pl.when(cnt > 0)
    def _():
        zeros = jnp.zeros((L,), jnp.int32)
        plist[pl.ds(cnt, L)] = plsc.load_gather(plist, [zeros])
        tlist[pl.ds(cnt, L)] = plsc.load_gather(tlist, [zeros])

    ngroups = (cnt + L - 1) // L

    def drain(g_i, carry):
        pos16 = plist[pl.ds(g_i * L, L)]
        tgt16 = tlist[pl.ds(g_i * L, L)]
        pltpu.async_copy(vals_hbm.at[pos16], rows, sem).wait()
        pltpu.async_copy(rows, out_ref.at[tgt16], sem).wait()
        return carry

    lax.fori_loop(0, ngroups, drain, 0)


def kernel(x, idx, vals):
    out = jax.new_ref(x)
    _sc_index_put(out, idx.astype(jnp.int32), vals)
    return out[...]


# speedup vs baseline: 7.4722x; 4.3329x over previous
import functools

import jax
import jax.numpy as jnp
from jax import lax
from jax.experimental import pallas as pl
from jax.experimental.pallas import tpu as pltpu
from jax.experimental.pallas import tpu_sc as plsc

L = 16
NC, NS = 2, 16
NW = NC * NS
R = 1_000_000
D = 64
B = 16384
CW = 512
CPW = 61
PPW = CPW * CW
TAIL = R - 1953 * CW
MAXP = R - 31 * PPW
NCHUNK = B // L

_mesh = plsc.VectorSubcoreMesh(core_axis_name="c", subcore_axis_name="s")


@functools.partial(
    pl.kernel,
    out_type=jax.ShapeDtypeStruct((D, R), jnp.float32),
    mesh=_mesh,
    compiler_params=pltpu.CompilerParams(
        needs_layout_passes=False, use_tc_tiling_on_sc=True),
    scratch_types=[
        pltpu.VMEM((B,), jnp.int32),
        pltpu.VMEM((MAXP,), jnp.int32),
        pltpu.VMEM((CW + L,), jnp.int32),
        pltpu.VMEM((CW + L,), jnp.int32),
        pltpu.VMEM((D, CW), jnp.float32),
        pltpu.VMEM((D, CW), jnp.float32),
        pltpu.VMEM((L, 128), jnp.float32),
        pltpu.VMEM((D, TAIL), jnp.float32),
        pltpu.SemaphoreType.DMA,
        pltpu.SemaphoreType.DMA,
        pltpu.SemaphoreType.DMA,
    ],
)
def _sc_index_put_fused(xt_hbm, idx_hbm, vals_pad_hbm, out_ref, idx_v, marker,
                        hcol, hpos, blk0, blk1, gbuf, tblk, lsem, ssem, gsem):
    wid = lax.axis_index("s") * NC + lax.axis_index("c")
    base = wid * PPW
    npos = jnp.where(wid == NW - 1, MAXP, PPW)
    lane = lax.iota(jnp.int32, L)

    pltpu.sync_copy(idx_hbm, idx_v)

    def clear(i, carry):
        marker[pl.ds(i * L, L)] = jnp.full((L,), -1, jnp.int32)
        return carry

    lax.fori_loop(0, (MAXP + L - 1) // L, clear, 0)

    def p1(c, carry):
        i16 = idx_v[pl.ds(c * L, L)]
        local = i16 - base
        member = (i16 >= base) & (local < npos)
        plsc.store_scatter(marker, [local], c * L + lane, mask=member)
        return carry

    lax.fori_loop(0, NCHUNK, p1, 0)

    def patch(blk, moff, width):
        def scan(k, nh):
            m16 = marker[pl.ds(moff + k * L, L)]
            hit = m16 >= 0
            plsc.store_compressed(hcol.at[pl.ds(nh, L)], k * L + lane,
                                  mask=hit)
            plsc.store_compressed(hpos.at[pl.ds(nh, L)], m16, mask=hit)
            return nh + jnp.sum(hit.astype(jnp.int32))

        nhits = lax.fori_loop(0, width // L, scan, jnp.int32(0))

        def group(g, carry):
            gbase = g * L
            m16 = hpos[pl.ds(gbase, L)]
            valid = lane < (nhits - gbase)
            m16 = jnp.where(valid, m16, jnp.broadcast_to(m16[0], (L,)))
            pltpu.async_copy(vals_pad_hbm.at[m16], gbuf, gsem).wait()

            def one(l, carry2):
                col = hcol[pl.ds(gbase + l, L)][0]
                cvec = jnp.broadcast_to(col, (L,)).astype(jnp.int32)
                lvec = jnp.broadcast_to(l, (L,)).astype(jnp.int32)
                for k in range(D // L):
                    v = plsc.load_gather(gbuf, [lvec, k * L + lane])
                    plsc.store_scatter(blk, [k * L + lane, cvec], v)
                return carry2

            lax.fori_loop(0, jnp.minimum(nhits - gbase, L), one, 0)
            return carry

        lax.fori_loop(0, (nhits + L - 1) // L, group, 0)

    def start_load(c, blk):
        off = base + c * CW
        pltpu.async_copy(xt_hbm.at[:, pl.ds(off, CW)], blk, lsem)

    def wait_load(blk):
        pltpu.make_async_copy(xt_hbm.at[:, pl.ds(0, CW)], blk, lsem).wait()

    start_load(0, blk0)

    def step(c, carry):
        off = base + c * CW

        @pl.when(c >= 1)
        def _():
            pltpu.make_async_copy(blk0, out_ref.at[:, pl.ds(0, CW)],
                                  ssem).wait()

        @pl.when(c % 2 == 0)
        def _():
            wait_load(blk0)

            @pl.when(c + 1 < CPW)
            def _():
                start_load(c + 1, blk1)

            patch(blk0, c * CW, CW)
            pltpu.async_copy(blk0, out_ref.at[:, pl.ds(off, CW)], ssem)

        @pl.when(c % 2 == 1)
        def _():
            wait_load(blk1)

            @pl.when(c + 1 < CPW)
            def _():
                start_load(c + 1, blk0)

            patch(blk1, c * CW, CW)
            pltpu.async_copy(blk1, out_ref.at[:, pl.ds(off, CW)], ssem)

        return carry

    lax.fori_loop(0, CPW, step, 0)
    pltpu.make_async_copy(blk0, out_ref.at[:, pl.ds(0, CW)], ssem).wait()

    @pl.when(wid == NW - 1)
    def _():
        off = 1952 * CW
        pltpu.sync_copy(xt_hbm.at[:, pl.ds(off, CW)], blk0)
        patch(blk0, off - base, CW)
        pltpu.sync_copy(blk0, out_ref.at[:, pl.ds(off, CW)])

        toff = 1953 * CW
        pltpu.sync_copy(xt_hbm.at[:, pl.ds(toff, TAIL)], tblk)
        patch(tblk, toff - base, TAIL)
        pltpu.sync_copy(tblk, out_ref.at[:, pl.ds(toff, TAIL)])


def kernel(x, idx, vals):
    vals_pad = jnp.pad(vals, ((0, 0), (0, 128 - D)))
    out_t = _sc_index_put_fused(x.T, idx.astype(jnp.int32), vals_pad)
    return out_t.T
